# Initial kernel scaffold; baseline (speedup 1.0000x reference)
import jax
import jax.numpy as jnp
from jax.experimental import pallas as pl


def kernel(edge_attr, dst_ports, tcp_flags, edge_index, batch, y, port_emb, tcp_emb, W1, b1, W2, b2, Wc1, bc1, Wc2, bc2):
    raise NotImplementedError("write your pallas kernel here")


# speedup vs baseline: 6.8166x; 6.8166x over previous
import functools

import jax
import jax.numpy as jnp
from jax import lax
from jax.experimental import pallas as pl
from jax.experimental.pallas import tpu as pltpu
from jax.experimental.pallas import tpu_sc as plsc

E = 320000
N = 10000
G = 64
H = 128
CHUNK = 128
NCHUNKS = E // CHUNK
NC, NS = 2, 16
NW = NC * NS
W_BASE, W_EXTRA = divmod(NCHUNKS, NW)
S_BASE, S_EXTRA = divmod(NCHUNKS, NS)
NP = 10240
RPS = NP // NS
ZR = 128
BE = 2000
BN = 2000

_mesh = plsc.VectorSubcoreMesh(core_axis_name="c", subcore_axis_name="s")


def _zero_rows(buf, nrows):
    def row(i, _):
        for j in range(buf.shape[1] // 16):
            buf[i, pl.ds(j * 16, 16)] = jnp.zeros((16,), jnp.float32)
        return 0
    lax.fori_loop(0, nrows, row, 0)


def _k1_body(port_pad, dst_ports, out, pidx, rows):
    wid = lax.axis_index("s") * NC + lax.axis_index("c")
    nmine = W_BASE + jnp.where(wid < W_EXTRA, 1, 0)

    def chunk(j, _):
        base = (j * NW + wid) * CHUNK
        pltpu.sync_copy(dst_ports.at[pl.ds(base, CHUNK)], pidx)
        pltpu.sync_copy(port_pad.at[pidx], rows)
        pltpu.sync_copy(rows, out.at[pl.ds(base, CHUNK)])
        return 0
    lax.fori_loop(0, nmine, chunk, 0)


_k1 = functools.partial(
    pl.kernel, _k1_body, mesh=_mesh,
    out_type=jax.ShapeDtypeStruct((E, H), jnp.float32),
    scratch_types=[
        pltpu.VMEM((CHUNK,), jnp.int32),
        pltpu.VMEM((CHUNK, H), jnp.float32),
    ],
)()


def _k2_body(attr_ref, pe_ref, tcp_ref, w1a_ref, w1b_ref, w1c_ref, b1_ref,
             temb_ref, payA_ref, payB_ref):
    attr = attr_ref[...]
    pe = pe_ref[:, :14]
    t = tcp_ref[0]
    ohT = (lax.broadcasted_iota(jnp.int32, (256, BE), 0) == t
           ).astype(jnp.float32)
    te = lax.dot_general(ohT, temb_ref[...], (((0,), (0,)), ((), ())),
                         precision=lax.Precision.HIGHEST,
                         preferred_element_type=jnp.float32)
    z = (jnp.dot(attr, w1a_ref[...], preferred_element_type=jnp.float32)
         + jnp.dot(pe, w1b_ref[...], preferred_element_type=jnp.float32)
         + jnp.dot(te, w1c_ref[...], preferred_element_type=jnp.float32)
         + b1_ref[...])
    payA_ref[...] = jnp.maximum(z, 0.0)
    payB_ref[...] = jnp.concatenate(
        [attr, pe, te, jnp.ones((BE, 1), jnp.float32),
         jnp.zeros((BE, 95), jnp.float32)], axis=1)


def _run_k2(edge_attr, pe128, tcp3d, w1a, w1b, w1c, b1r, tcp_emb):
    return pl.pallas_call(
        _k2_body,
        grid=(E // BE,),
        in_specs=[
            pl.BlockSpec((BE, 16), lambda i: (i, 0)),
            pl.BlockSpec((BE, H), lambda i: (i, 0)),
            pl.BlockSpec((1, 1, BE), lambda i: (i, 0, 0)),
            pl.BlockSpec((16, H), lambda i: (0, 0)),
            pl.BlockSpec((14, H), lambda i: (0, 0)),
            pl.BlockSpec((2, H), lambda i: (0, 0)),
            pl.BlockSpec((1, H), lambda i: (0, 0)),
            pl.BlockSpec((256, 2), lambda i: (0, 0)),
        ],
        out_specs=(pl.BlockSpec((BE, H), lambda i: (i, 0)),
                   pl.BlockSpec((BE, H), lambda i: (i, 0))),
        out_shape=(jax.ShapeDtypeStruct((E, H), jnp.float32),
                   jax.ShapeDtypeStruct((E, H), jnp.float32)),
    )(edge_attr, pe128, tcp3d, w1a, w1b, w1c, b1r, tcp_emb)


def _k3_body(payA, payB, dst, out, accum, idx, pbuf, zbuf):
    c = lax.axis_index("c")
    s = lax.axis_index("s")
    nmine = S_BASE + jnp.where(s < S_EXTRA, 1, 0)

    _zero_rows(zbuf, ZR)
    row0 = s * RPS
    for k in range(RPS // ZR):
        pltpu.sync_copy(zbuf, accum.at[pl.ds(row0 + k * ZR, ZR)])
    plsc.subcore_barrier()

    def chunk(j, _):
        base = (j * NS + s) * CHUNK
        pltpu.sync_copy(dst.at[pl.ds(base, CHUNK)], idx)

        @pl.when(c == 0)
        def _():
            pltpu.sync_copy(payA.at[pl.ds(base, CHUNK)], pbuf)

        @pl.when(c == 1)
        def _():
            pltpu.sync_copy(payB.at[pl.ds(base, CHUNK)], pbuf)

        pltpu.sync_copy(pbuf, accum.at[idx], add=True)
        return 0
    lax.fori_loop(0, nmine, chunk, 0)
    plsc.subcore_barrier()

    pltpu.sync_copy(accum.at[pl.ds(row0, RPS)],
                    out.at[c, pl.ds(row0, RPS)])


_k3 = functools.partial(
    pl.kernel, _k3_body, mesh=_mesh,
    out_type=jax.ShapeDtypeStruct((NC, NP, H), jnp.float32),
    scratch_types=[
        pltpu.VMEM_SHARED((NP, H), jnp.float32),
        pltpu.VMEM((CHUNK,), jnp.int32),
        pltpu.VMEM((CHUNK, H), jnp.float32),
        pltpu.VMEM((ZR, H), jnp.float32),
    ],
)()


def _k4_body(agg_ref, w1a_ref, w1b_ref, w1c_ref, b1_ref, w2_ref, b2_ref,
             m_ref, invd_ref):
    s1 = agg_ref[0]
    lin = agg_ref[1]
    deg = lin[:, 32:33]
    maxdeg = jnp.maximum(deg, 1.0)
    degf = deg + 1.0
    zl = (jnp.dot(lin[:, 0:16], w1a_ref[...],
                  preferred_element_type=jnp.float32)
          + jnp.dot(lin[:, 16:30], w1b_ref[...],
                    preferred_element_type=jnp.float32)
          + jnp.dot(lin[:, 30:32], w1c_ref[...],
                    preferred_element_type=jnp.float32))
    h1l = jnp.maximum(zl / maxdeg + b1_ref[...], 0.0)
    t = (s1 + h1l) / degf
    m_ref[...] = (jnp.dot(t, w2_ref[...], preferred_element_type=jnp.float32)
                  + b2_ref[...])
    invd_ref[...] = jnp.broadcast_to(1.0 / degf, invd_ref.shape)


def _run_k4(agg, w1a, w1b, w1c, b1r, w2, b2r):
    return pl.pallas_call(
        _k4_body,
        grid=(N // BN,),
        in_specs=[
            pl.BlockSpec((NC, BN, H), lambda i: (0, i, 0)),
            pl.BlockSpec((16, H), lambda i: (0, 0)),
            pl.BlockSpec((14, H), lambda i: (0, 0)),
            pl.BlockSpec((2, H), lambda i: (0, 0)),
            pl.BlockSpec((1, H), lambda i: (0, 0)),
            pl.BlockSpec((H, H), lambda i: (0, 0)),
            pl.BlockSpec((1, H), lambda i: (0, 0)),
        ],
        out_specs=(pl.BlockSpec((BN, H), lambda i: (i, 0)),
                   pl.BlockSpec((BN, 8), lambda i: (i, 0))),
        out_shape=(jax.ShapeDtypeStruct((N, H), jnp.float32),
                   jax.ShapeDtypeStruct((N, 8), jnp.float32)),
    )(agg, w1a, w1b, w1c, b1r, w2, b2r)


def _k5_body(v, src, dst, part_out, accum, sidx, didx, buf, zbuf):
    c = lax.axis_index("c")
    s = lax.axis_index("s")
    wid = s * NC + c
    nmine = W_BASE + jnp.where(wid < W_EXTRA, 1, 0)

    _zero_rows(zbuf, ZR)
    row0 = s * RPS
    for k in range(RPS // ZR):
        pltpu.sync_copy(zbuf, accum.at[pl.ds(row0 + k * ZR, ZR)])
    plsc.subcore_barrier()

    def chunk(j, _):
        base = (j * NW + wid) * CHUNK
        pltpu.sync_copy(src.at[pl.ds(base, CHUNK)], sidx)
        pltpu.sync_copy(dst.at[pl.ds(base, CHUNK)], didx)
        pltpu.sync_copy(v.at[sidx], buf)
        pltpu.sync_copy(buf, accum.at[didx], add=True)
        return 0
    lax.fori_loop(0, nmine, chunk, 0)
    plsc.subcore_barrier()

    pltpu.sync_copy(accum.at[pl.ds(row0, RPS)],
                    part_out.at[c, pl.ds(row0, RPS)])


_k5 = functools.partial(
    pl.kernel, _k5_body, mesh=_mesh,
    out_type=jax.ShapeDtypeStruct((NC, NP, H), jnp.float32),
    scratch_types=[
        pltpu.VMEM_SHARED((NP, H), jnp.float32),
        pltpu.VMEM((CHUNK,), jnp.int32),
        pltpu.VMEM((CHUNK,), jnp.int32),
        pltpu.VMEM((CHUNK, H), jnp.float32),
        pltpu.VMEM((ZR, H), jnp.float32),
    ],
)()


def _k6_body(p_ref, m_ref, v_ref, invd_ref, out_ref):
    out_ref[...] = m_ref[...] + (p_ref[0] + p_ref[1] + v_ref[...]) \
        * invd_ref[:, 0:1]


def _run_k6(part, m, v, invd):
    return pl.pallas_call(
        _k6_body,
        grid=(N // BN,),
        in_specs=[
            pl.BlockSpec((NC, BN, H), lambda i: (0, i, 0)),
            pl.BlockSpec((BN, H), lambda i: (i, 0)),
            pl.BlockSpec((BN, H), lambda i: (i, 0)),
            pl.BlockSpec((BN, 8), lambda i: (i, 0)),
        ],
        out_specs=pl.BlockSpec((BN, H), lambda i: (i, 0)),
        out_shape=jax.ShapeDtypeStruct((N, H), jnp.float32),
    )(part, m, v, invd)


def _k7_body(p_ref, m_ref, v_ref, invd_ref, batch_ref, wc1_ref, bc1_ref,
             wc2_ref, bc2_ref, out_ref, acc, gcnt):
    i = pl.program_id(0)

    @pl.when(i == 0)
    def _():
        acc[...] = jnp.zeros_like(acc)
        gcnt[...] = jnp.zeros_like(gcnt)

    x3 = m_ref[...] + (p_ref[0] + p_ref[1] + v_ref[...]) \
        * invd_ref[:, 0:1]
    gids = lax.broadcasted_iota(jnp.int32, (G, BN), 0)
    mask = (gids == batch_ref[0]).astype(jnp.float32)
    acc[...] += jnp.dot(mask, x3, preferred_element_type=jnp.float32)
    gcnt[...] += jnp.broadcast_to(
        jnp.sum(mask, axis=1, keepdims=True), gcnt.shape)

    @pl.when(i == pl.num_programs(0) - 1)
    def _():
        pooled = acc[...] / jnp.maximum(gcnt[...], 1.0)
        h = jnp.maximum(
            jnp.dot(pooled, wc1_ref[...], preferred_element_type=jnp.float32)
            + bc1_ref[...], 0.0)
        out_ref[...] = (jnp.dot(h, wc2_ref[...],
                                preferred_element_type=jnp.float32)
                        + bc2_ref[...])


def _run_k7(part, m, v, invd, batch3d, wc1, bc1r, wc2, bc2r):
    return pl.pallas_call(
        _k7_body,
        grid=(N // BN,),
        in_specs=[
            pl.BlockSpec((NC, BN, H), lambda i: (0, i, 0)),
            pl.BlockSpec((BN, H), lambda i: (i, 0)),
            pl.BlockSpec((BN, H), lambda i: (i, 0)),
            pl.BlockSpec((BN, 8), lambda i: (i, 0)),
            pl.BlockSpec((1, 1, BN), lambda i: (i, 0, 0)),
            pl.BlockSpec((H, H), lambda i: (0, 0)),
            pl.BlockSpec((1, H), lambda i: (0, 0)),
            pl.BlockSpec((H, 10), lambda i: (0, 0)),
            pl.BlockSpec((1, 10), lambda i: (0, 0)),
        ],
        out_specs=pl.BlockSpec((G, 10), lambda i: (0, 0)),
        out_shape=jax.ShapeDtypeStruct((G, 10), jnp.float32),
        scratch_shapes=[pltpu.VMEM((G, H), jnp.float32),
                        pltpu.VMEM((G, H), jnp.float32)],
    )(part, m, v, invd, batch3d, wc1, bc1r, wc2, bc2r)


def kernel(edge_attr, dst_ports, tcp_flags, edge_index, batch, y,
           port_emb, tcp_emb, W1, b1, W2, b2, Wc1, bc1, Wc2, bc2):
    port_pad = jnp.pad(port_emb, ((0, 0), (0, H - 14)))
    src = edge_index[0]
    dst = edge_index[1]
    w1a = W1[:16]
    w1b = W1[16:30]
    w1c = W1[30:32]
    b1r = b1.reshape(1, H)
    b2r = b2.reshape(1, H)
    bc1r = bc1.reshape(1, H)
    bc2r = bc2.reshape(1, 10)
    tcp3d = tcp_flags.astype(jnp.int32).reshape(E // BE, 1, BE)
    batch3d = batch.astype(jnp.int32).reshape(N // BN, 1, BN)

    pe128 = _k1(port_pad, dst_ports)
    payA, payB = _run_k2(edge_attr, pe128, tcp3d, w1a, w1b, w1c, b1r, tcp_emb)
    agg = _k3(payA, payB, dst)
    m, invd = _run_k4(agg, w1a, w1b, w1c, b1r, W2, b2r)
    p1 = _k5(m, src, dst)
    x2 = _run_k6(p1, m, m, invd)
    p2 = _k5(x2, src, dst)
    return _run_k7(p2, m, x2, invd, batch3d, Wc1, bc1r, Wc2, bc2r)


# speedup vs baseline: 8.8701x; 1.3013x over previous
import functools

import jax
import jax.numpy as jnp
from jax import lax
from jax.experimental import pallas as pl
from jax.experimental.pallas import tpu as pltpu
from jax.experimental.pallas import tpu_sc as plsc

E = 320000
N = 10000
G = 64
H = 128
CHUNK = 128
NCHUNKS = E // CHUNK
NC, NS = 2, 16
NW = NC * NS
W_BASE, W_EXTRA = divmod(NCHUNKS, NW)
S_BASE, S_EXTRA = divmod(NCHUNKS, NS)
NP = 10240
RPS = NP // NS
ZR = 64
BE = 2000
BN = 2000

_mesh = plsc.VectorSubcoreMesh(core_axis_name="c", subcore_axis_name="s")


def _zero_rows(buf, nrows):
    def row(i, _):
        for j in range(buf.shape[1] // 16):
            buf[i, pl.ds(j * 16, 16)] = jnp.zeros((16,), jnp.float32)
        return 0
    lax.fori_loop(0, nrows, row, 0)


def _k1_body(port_pad, dst_ports, out, pidx, rows):
    wid = lax.axis_index("s") * NC + lax.axis_index("c")
    nmine = W_BASE + jnp.where(wid < W_EXTRA, 1, 0)

    def chunk(j, _):
        base = (j * NW + wid) * CHUNK
        pltpu.sync_copy(dst_ports.at[pl.ds(base, CHUNK)], pidx)
        pltpu.sync_copy(port_pad.at[pidx], rows)
        pltpu.sync_copy(rows, out.at[pl.ds(base, CHUNK)])
        return 0
    lax.fori_loop(0, nmine, chunk, 0)


_k1 = functools.partial(
    pl.kernel, _k1_body, mesh=_mesh,
    out_type=jax.ShapeDtypeStruct((E, H), jnp.float32),
    scratch_types=[
        pltpu.VMEM((CHUNK,), jnp.int32),
        pltpu.VMEM((CHUNK, H), jnp.float32),
    ],
)()


def _k2_body(attr_ref, pe_ref, tcp_ref, w1a_ref, w1b_ref, w1c_ref, b1_ref,
             temb_ref, payA_ref, payB_ref):
    attr = attr_ref[...]
    pe = pe_ref[:, :14]
    t = tcp_ref[0]
    ohT = (lax.broadcasted_iota(jnp.int32, (256, BE), 0) == t
           ).astype(jnp.float32)
    te = lax.dot_general(ohT, temb_ref[...], (((0,), (0,)), ((), ())),
                         precision=lax.Precision.HIGHEST,
                         preferred_element_type=jnp.float32)
    z = (jnp.dot(attr, w1a_ref[...], preferred_element_type=jnp.float32)
         + jnp.dot(pe, w1b_ref[...], preferred_element_type=jnp.float32)
         + jnp.dot(te, w1c_ref[...], preferred_element_type=jnp.float32)
         + b1_ref[...])
    payA_ref[...] = jnp.maximum(z, 0.0)
    payB_ref[...] = jnp.concatenate(
        [attr, pe, te, jnp.ones((BE, 1), jnp.float32),
         jnp.zeros((BE, 95), jnp.float32)], axis=1)


def _run_k2(edge_attr, pe128, tcp3d, w1a, w1b, w1c, b1r, tcp_emb):
    return pl.pallas_call(
        _k2_body,
        grid=(E // BE,),
        in_specs=[
            pl.BlockSpec((BE, 16), lambda i: (i, 0)),
            pl.BlockSpec((BE, H), lambda i: (i, 0)),
            pl.BlockSpec((1, 1, BE), lambda i: (i, 0, 0)),
            pl.BlockSpec((16, H), lambda i: (0, 0)),
            pl.BlockSpec((14, H), lambda i: (0, 0)),
            pl.BlockSpec((2, H), lambda i: (0, 0)),
            pl.BlockSpec((1, H), lambda i: (0, 0)),
            pl.BlockSpec((256, 2), lambda i: (0, 0)),
        ],
        out_specs=(pl.BlockSpec((BE, H), lambda i: (i, 0)),
                   pl.BlockSpec((BE, H), lambda i: (i, 0))),
        out_shape=(jax.ShapeDtypeStruct((E, H), jnp.float32),
                   jax.ShapeDtypeStruct((E, H), jnp.float32)),
    )(edge_attr, pe128, tcp3d, w1a, w1b, w1c, b1r, tcp_emb)


def _k3_body(payA, payB, dst, out, accum, i0, i1, b0, b1, zbuf, sem0, sem1):
    c = lax.axis_index("c")
    s = lax.axis_index("s")
    nmine = S_BASE + jnp.where(s < S_EXTRA, 1, 0)
    npairs = nmine // 2
    tail = nmine - npairs * 2

    _zero_rows(zbuf, ZR)
    row0 = s * RPS
    for k in range(RPS // ZR):
        pltpu.sync_copy(zbuf, accum.at[pl.ds(row0 + k * ZR, ZR)])
    plsc.subcore_barrier()

    def cbase(k):
        return (k * NS + s) * CHUNK

    def start(k, buf, sem):
        @pl.when(c == 0)
        def _():
            pltpu.async_copy(payA.at[pl.ds(cbase(k), CHUNK)], buf, sem)

        @pl.when(c == 1)
        def _():
            pltpu.async_copy(payB.at[pl.ds(cbase(k), CHUNK)], buf, sem)

    def wait(buf, sem):
        pltpu.make_async_copy(payA.at[pl.ds(0, CHUNK)], buf, sem).wait()

    start(0, b0, sem0)

    def pair(j, _):
        k0 = 2 * j
        start(k0 + 1, b1, sem1)
        wait(b0, sem0)
        pltpu.sync_copy(dst.at[pl.ds(cbase(k0), CHUNK)], i0)
        pltpu.sync_copy(b0, accum.at[i0], add=True)

        @pl.when(k0 + 2 < nmine)
        def _():
            start(k0 + 2, b0, sem0)

        wait(b1, sem1)
        pltpu.sync_copy(dst.at[pl.ds(cbase(k0 + 1), CHUNK)], i1)
        pltpu.sync_copy(b1, accum.at[i1], add=True)
        return 0
    lax.fori_loop(0, npairs, pair, 0)

    @pl.when(tail == 1)
    def _():
        k = npairs * 2
        wait(b0, sem0)
        pltpu.sync_copy(dst.at[pl.ds(cbase(k), CHUNK)], i0)
        pltpu.sync_copy(b0, accum.at[i0], add=True)
    plsc.subcore_barrier()

    pltpu.sync_copy(accum.at[pl.ds(row0, RPS)],
                    out.at[c, pl.ds(row0, RPS)])


_k3 = functools.partial(
    pl.kernel, _k3_body, mesh=_mesh,
    out_type=jax.ShapeDtypeStruct((NC, NP, H), jnp.float32),
    scratch_types=[
        pltpu.VMEM_SHARED((NP, H), jnp.float32),
        pltpu.VMEM((CHUNK,), jnp.int32),
        pltpu.VMEM((CHUNK,), jnp.int32),
        pltpu.VMEM((CHUNK, H), jnp.float32),
        pltpu.VMEM((CHUNK, H), jnp.float32),
        pltpu.VMEM((ZR, H), jnp.float32),
        pltpu.SemaphoreType.DMA,
        pltpu.SemaphoreType.DMA,
    ],
)()


def _k4_body(agg_ref, w1a_ref, w1b_ref, w1c_ref, b1_ref, w2_ref,
             b2_ref, m_ref, invd_ref):
    s1 = agg_ref[0]
    lin = agg_ref[1]
    deg = lin[:, 32:33]
    maxdeg = jnp.maximum(deg, 1.0)
    degf = deg + 1.0
    zl = (jnp.dot(lin[:, 0:16], w1a_ref[...],
                  preferred_element_type=jnp.float32)
          + jnp.dot(lin[:, 16:30], w1b_ref[...],
                    preferred_element_type=jnp.float32)
          + jnp.dot(lin[:, 30:32], w1c_ref[...],
                    preferred_element_type=jnp.float32))
    h1l = jnp.maximum(zl / maxdeg + b1_ref[...], 0.0)
    t = (s1 + h1l) / degf
    m_ref[...] = (jnp.dot(t, w2_ref[...], preferred_element_type=jnp.float32)
                  + b2_ref[...])
    invd_ref[...] = jnp.broadcast_to(1.0 / degf, invd_ref.shape)


def _run_k4(agg, w1a, w1b, w1c, b1r, w2, b2r):
    return pl.pallas_call(
        _k4_body,
        grid=(N // BN,),
        in_specs=[
            pl.BlockSpec((NC, BN, H), lambda i: (0, i, 0)),
            pl.BlockSpec((16, H), lambda i: (0, 0)),
            pl.BlockSpec((14, H), lambda i: (0, 0)),
            pl.BlockSpec((2, H), lambda i: (0, 0)),
            pl.BlockSpec((1, H), lambda i: (0, 0)),
            pl.BlockSpec((H, H), lambda i: (0, 0)),
            pl.BlockSpec((1, H), lambda i: (0, 0)),
        ],
        out_specs=(pl.BlockSpec((BN, H), lambda i: (i, 0)),
                   pl.BlockSpec((BN, 8), lambda i: (i, 0))),
        out_shape=(jax.ShapeDtypeStruct((N, H), jnp.float32),
                   jax.ShapeDtypeStruct((N, 8), jnp.float32)),
    )(agg, w1a, w1b, w1c, b1r, w2, b2r)


def _k5_body(v, src, dst, part_out, accum, si0, si1, di, b0, b1, zbuf,
             sem0, sem1):
    c = lax.axis_index("c")
    s = lax.axis_index("s")
    wid = s * NC + c
    nmine = W_BASE + jnp.where(wid < W_EXTRA, 1, 0)
    npairs = nmine // 2
    tail = nmine - npairs * 2

    _zero_rows(zbuf, ZR)
    row0 = s * RPS
    for k in range(RPS // ZR):
        pltpu.sync_copy(zbuf, accum.at[pl.ds(row0 + k * ZR, ZR)])
    plsc.subcore_barrier()

    def cbase(k):
        return (k * NW + wid) * CHUNK

    def startg(k, si, buf, sem):
        pltpu.sync_copy(src.at[pl.ds(cbase(k), CHUNK)], si)
        pltpu.async_copy(v.at[si], buf, sem)

    def waitg(buf, sem):
        pltpu.make_async_copy(v.at[si0], buf, sem).wait()

    startg(0, si0, b0, sem0)

    def pair(j, _):
        k0 = 2 * j
        startg(k0 + 1, si1, b1, sem1)
        waitg(b0, sem0)
        pltpu.sync_copy(dst.at[pl.ds(cbase(k0), CHUNK)], di)
        pltpu.sync_copy(b0, accum.at[di], add=True)

        @pl.when(k0 + 2 < nmine)
        def _():
            startg(k0 + 2, si0, b0, sem0)

        waitg(b1, sem1)
        pltpu.sync_copy(dst.at[pl.ds(cbase(k0 + 1), CHUNK)], di)
        pltpu.sync_copy(b1, accum.at[di], add=True)
        return 0
    lax.fori_loop(0, npairs, pair, 0)

    @pl.when(tail == 1)
    def _():
        k = npairs * 2
        waitg(b0, sem0)
        pltpu.sync_copy(dst.at[pl.ds(cbase(k), CHUNK)], di)
        pltpu.sync_copy(b0, accum.at[di], add=True)
    plsc.subcore_barrier()

    pltpu.sync_copy(accum.at[pl.ds(row0, RPS)],
                    part_out.at[c, pl.ds(row0, RPS)])


_k5 = functools.partial(
    pl.kernel, _k5_body, mesh=_mesh,
    out_type=jax.ShapeDtypeStruct((NC, NP, H), jnp.float32),
    scratch_types=[
        pltpu.VMEM_SHARED((NP, H), jnp.float32),
        pltpu.VMEM((CHUNK,), jnp.int32),
        pltpu.VMEM((CHUNK,), jnp.int32),
        pltpu.VMEM((CHUNK,), jnp.int32),
        pltpu.VMEM((CHUNK, H), jnp.float32),
        pltpu.VMEM((CHUNK, H), jnp.float32),
        pltpu.VMEM((ZR, H), jnp.float32),
        pltpu.SemaphoreType.DMA,
        pltpu.SemaphoreType.DMA,
    ],
)()


def _k6_body(p_ref, m_ref, v_ref, invd_ref, out_ref):
    out_ref[...] = m_ref[...] + (p_ref[0] + p_ref[1] + v_ref[...]) \
        * invd_ref[:, 0:1]


def _run_k6(part, m, v, invd):
    return pl.pallas_call(
        _k6_body,
        grid=(N // BN,),
        in_specs=[
            pl.BlockSpec((NC, BN, H), lambda i: (0, i, 0)),
            pl.BlockSpec((BN, H), lambda i: (i, 0)),
            pl.BlockSpec((BN, H), lambda i: (i, 0)),
            pl.BlockSpec((BN, 8), lambda i: (i, 0)),
        ],
        out_specs=pl.BlockSpec((BN, H), lambda i: (i, 0)),
        out_shape=jax.ShapeDtypeStruct((N, H), jnp.float32),
    )(part, m, v, invd)


def _k7_body(p_ref, m_ref, v_ref, invd_ref, batch_ref, wc1_ref, bc1_ref,
             wc2_ref, bc2_ref, out_ref, acc, gcnt):
    i = pl.program_id(0)

    @pl.when(i == 0)
    def _():
        acc[...] = jnp.zeros_like(acc)
        gcnt[...] = jnp.zeros_like(gcnt)

    x3 = m_ref[...] + (p_ref[0] + p_ref[1] + v_ref[...]) \
        * invd_ref[:, 0:1]
    gids = lax.broadcasted_iota(jnp.int32, (G, BN), 0)
    mask = (gids == batch_ref[0]).astype(jnp.float32)
    acc[...] += jnp.dot(mask, x3, preferred_element_type=jnp.float32)
    gcnt[...] += jnp.broadcast_to(
        jnp.sum(mask, axis=1, keepdims=True), gcnt.shape)

    @pl.when(i == pl.num_programs(0) - 1)
    def _():
        pooled = acc[...] / jnp.maximum(gcnt[...], 1.0)
        h = jnp.maximum(
            jnp.dot(pooled, wc1_ref[...], preferred_element_type=jnp.float32)
            + bc1_ref[...], 0.0)
        out_ref[...] = (jnp.dot(h, wc2_ref[...],
                                preferred_element_type=jnp.float32)
                        + bc2_ref[...])


def _run_k7(part, m, v, invd, batch3d, wc1, bc1r, wc2, bc2r):
    return pl.pallas_call(
        _k7_body,
        grid=(N // BN,),
        in_specs=[
            pl.BlockSpec((NC, BN, H), lambda i: (0, i, 0)),
            pl.BlockSpec((BN, H), lambda i: (i, 0)),
            pl.BlockSpec((BN, H), lambda i: (i, 0)),
            pl.BlockSpec((BN, 8), lambda i: (i, 0)),
            pl.BlockSpec((1, 1, BN), lambda i: (i, 0, 0)),
            pl.BlockSpec((H, H), lambda i: (0, 0)),
            pl.BlockSpec((1, H), lambda i: (0, 0)),
            pl.BlockSpec((H, 10), lambda i: (0, 0)),
            pl.BlockSpec((1, 10), lambda i: (0, 0)),
        ],
        out_specs=pl.BlockSpec((G, 10), lambda i: (0, 0)),
        out_shape=jax.ShapeDtypeStruct((G, 10), jnp.float32),
        scratch_shapes=[pltpu.VMEM((G, H), jnp.float32),
                        pltpu.VMEM((G, H), jnp.float32)],
    )(part, m, v, invd, batch3d, wc1, bc1r, wc2, bc2r)


def kernel(edge_attr, dst_ports, tcp_flags, edge_index, batch, y,
           port_emb, tcp_emb, W1, b1, W2, b2, Wc1, bc1, Wc2, bc2):
    port_pad = jnp.pad(port_emb, ((0, 0), (0, H - 14)))
    src = edge_index[0]
    dst = edge_index[1]
    w1a = W1[:16]
    w1b = W1[16:30]
    w1c = W1[30:32]
    b1r = b1.reshape(1, H)
    b2r = b2.reshape(1, H)
    bc1r = bc1.reshape(1, H)
    bc2r = bc2.reshape(1, 10)
    tcp3d = tcp_flags.astype(jnp.int32).reshape(E // BE, 1, BE)
    batch3d = batch.astype(jnp.int32).reshape(N // BN, 1, BN)

    pe128 = _k1(port_pad, dst_ports)
    payA, payB = _run_k2(edge_attr, pe128, tcp3d, w1a, w1b, w1c, b1r, tcp_emb)
    agg = _k3(payA, payB, dst)
    m, invd = _run_k4(agg, w1a, w1b, w1c, b1r, W2, b2r)
    p1 = _k5(m, src, dst)
    x2 = _run_k6(p1, m, m, invd)
    p2 = _k5(x2, src, dst)
    return _run_k7(p2, m, x2, invd, batch3d, Wc1, bc1r, Wc2, bc2r)


# speedup vs baseline: 9.1733x; 1.0342x over previous
import functools

import jax
import jax.numpy as jnp
from jax import lax
from jax.experimental import pallas as pl
from jax.experimental.pallas import tpu as pltpu
from jax.experimental.pallas import tpu_sc as plsc

E = 320000
N = 10000
G = 64
H = 128
CHUNK = 128
NCHUNKS = E // CHUNK
NC, NS = 2, 16
NW = NC * NS
W_BASE, W_EXTRA = divmod(NCHUNKS, NW)
S_BASE, S_EXTRA = divmod(NCHUNKS, NS)
NP = 10240
RPS = NP // NS
ZR = 64
BE = 2000
BN = 2000

_mesh = plsc.VectorSubcoreMesh(core_axis_name="c", subcore_axis_name="s")


def _zero_rows(buf, nrows):
    def row(i, _):
        for j in range(buf.shape[1] // 16):
            buf[i, pl.ds(j * 16, 16)] = jnp.zeros((16,), jnp.float32)
        return 0
    lax.fori_loop(0, nrows, row, 0)


def _k1_body(port_pad, dst_ports, out, pi0, pi1, b0, b1, sem0, sem1):
    wid = lax.axis_index("s") * NC + lax.axis_index("c")
    nmine = W_BASE + jnp.where(wid < W_EXTRA, 1, 0)
    npairs = nmine // 2
    tail = nmine - npairs * 2

    def cbase(k):
        return (k * NW + wid) * CHUNK

    def startg(k, pi, buf, sem):
        pltpu.sync_copy(dst_ports.at[pl.ds(cbase(k), CHUNK)], pi)
        pltpu.async_copy(port_pad.at[pi], buf, sem)

    def waitg(buf, sem):
        pltpu.make_async_copy(port_pad.at[pi0], buf, sem).wait()

    startg(0, pi0, b0, sem0)

    def pair(j, _):
        k0 = 2 * j
        startg(k0 + 1, pi1, b1, sem1)
        waitg(b0, sem0)
        pltpu.sync_copy(b0, out.at[pl.ds(cbase(k0), CHUNK)])

        @pl.when(k0 + 2 < nmine)
        def _():
            startg(k0 + 2, pi0, b0, sem0)

        waitg(b1, sem1)
        pltpu.sync_copy(b1, out.at[pl.ds(cbase(k0 + 1), CHUNK)])
        return 0
    lax.fori_loop(0, npairs, pair, 0)

    @pl.when(tail == 1)
    def _():
        k = npairs * 2
        waitg(b0, sem0)
        pltpu.sync_copy(b0, out.at[pl.ds(cbase(k), CHUNK)])


_k1 = functools.partial(
    pl.kernel, _k1_body, mesh=_mesh,
    out_type=jax.ShapeDtypeStruct((E, H), jnp.float32),
    scratch_types=[
        pltpu.VMEM((CHUNK,), jnp.int32),
        pltpu.VMEM((CHUNK,), jnp.int32),
        pltpu.VMEM((CHUNK, H), jnp.float32),
        pltpu.VMEM((CHUNK, H), jnp.float32),
        pltpu.SemaphoreType.DMA,
        pltpu.SemaphoreType.DMA,
    ],
)()


def _k2_body(attr_ref, pe_ref, tcp_ref, w1a_ref, w1b_ref, w1c_ref, b1_ref,
             temb_ref, payA_ref, payB_ref):
    attr = attr_ref[...]
    pe = pe_ref[:, :14]
    t = tcp_ref[0]
    ohT = (lax.broadcasted_iota(jnp.int32, (256, BE), 0) == t
           ).astype(jnp.float32)
    te = lax.dot_general(ohT, temb_ref[...], (((0,), (0,)), ((), ())),
                         precision=lax.Precision.HIGHEST,
                         preferred_element_type=jnp.float32)
    z = (jnp.dot(attr, w1a_ref[...], preferred_element_type=jnp.float32)
         + jnp.dot(pe, w1b_ref[...], preferred_element_type=jnp.float32)
         + jnp.dot(te, w1c_ref[...], preferred_element_type=jnp.float32)
         + b1_ref[...])
    payA_ref[...] = jnp.maximum(z, 0.0)
    payB_ref[...] = jnp.concatenate(
        [attr, pe, te, jnp.ones((BE, 1), jnp.float32),
         jnp.zeros((BE, 95), jnp.float32)], axis=1)


def _run_k2(edge_attr, pe128, tcp3d, w1a, w1b, w1c, b1r, tcp_emb):
    return pl.pallas_call(
        _k2_body,
        grid=(E // BE,),
        in_specs=[
            pl.BlockSpec((BE, 16), lambda i: (i, 0)),
            pl.BlockSpec((BE, H), lambda i: (i, 0)),
            pl.BlockSpec((1, 1, BE), lambda i: (i, 0, 0)),
            pl.BlockSpec((16, H), lambda i: (0, 0)),
            pl.BlockSpec((14, H), lambda i: (0, 0)),
            pl.BlockSpec((2, H), lambda i: (0, 0)),
            pl.BlockSpec((1, H), lambda i: (0, 0)),
            pl.BlockSpec((256, 2), lambda i: (0, 0)),
        ],
        out_specs=(pl.BlockSpec((BE, H), lambda i: (i, 0)),
                   pl.BlockSpec((BE, H), lambda i: (i, 0))),
        out_shape=(jax.ShapeDtypeStruct((E, H), jnp.float32),
                   jax.ShapeDtypeStruct((E, H), jnp.float32)),
    )(edge_attr, pe128, tcp3d, w1a, w1b, w1c, b1r, tcp_emb)


def _k3_body(payA, payB, dst, out, accum, i0, i1, b0, b1, zbuf, sem0, sem1):
    c = lax.axis_index("c")
    s = lax.axis_index("s")
    nmine = S_BASE + jnp.where(s < S_EXTRA, 1, 0)
    npairs = nmine // 2
    tail = nmine - npairs * 2

    _zero_rows(zbuf, ZR)
    row0 = s * RPS
    for k in range(RPS // ZR):
        pltpu.sync_copy(zbuf, accum.at[pl.ds(row0 + k * ZR, ZR)])
    plsc.subcore_barrier()

    def cbase(k):
        return (k * NS + s) * CHUNK

    def start(k, buf, sem):
        @pl.when(c == 0)
        def _():
            pltpu.async_copy(payA.at[pl.ds(cbase(k), CHUNK)], buf, sem)

        @pl.when(c == 1)
        def _():
            pltpu.async_copy(payB.at[pl.ds(cbase(k), CHUNK)], buf, sem)

    def wait(buf, sem):
        pltpu.make_async_copy(payA.at[pl.ds(0, CHUNK)], buf, sem).wait()

    start(0, b0, sem0)

    def pair(j, _):
        k0 = 2 * j
        start(k0 + 1, b1, sem1)
        wait(b0, sem0)
        pltpu.sync_copy(dst.at[pl.ds(cbase(k0), CHUNK)], i0)
        pltpu.sync_copy(b0, accum.at[i0], add=True)

        @pl.when(k0 + 2 < nmine)
        def _():
            start(k0 + 2, b0, sem0)

        wait(b1, sem1)
        pltpu.sync_copy(dst.at[pl.ds(cbase(k0 + 1), CHUNK)], i1)
        pltpu.sync_copy(b1, accum.at[i1], add=True)
        return 0
    lax.fori_loop(0, npairs, pair, 0)

    @pl.when(tail == 1)
    def _():
        k = npairs * 2
        wait(b0, sem0)
        pltpu.sync_copy(dst.at[pl.ds(cbase(k), CHUNK)], i0)
        pltpu.sync_copy(b0, accum.at[i0], add=True)
    plsc.subcore_barrier()

    pltpu.sync_copy(accum.at[pl.ds(row0, RPS)],
                    out.at[c, pl.ds(row0, RPS)])


_k3 = functools.partial(
    pl.kernel, _k3_body, mesh=_mesh,
    out_type=jax.ShapeDtypeStruct((NC, NP, H), jnp.float32),
    scratch_types=[
        pltpu.VMEM_SHARED((NP, H), jnp.float32),
        pltpu.VMEM((CHUNK,), jnp.int32),
        pltpu.VMEM((CHUNK,), jnp.int32),
        pltpu.VMEM((CHUNK, H), jnp.float32),
        pltpu.VMEM((CHUNK, H), jnp.float32),
        pltpu.VMEM((ZR, H), jnp.float32),
        pltpu.SemaphoreType.DMA,
        pltpu.SemaphoreType.DMA,
    ],
)()


def _k4_body(agg_ref, w1a_ref, w1b_ref, w1c_ref, b1_ref, w2_ref,
             b2_ref, m_ref, invd_ref):
    s1 = agg_ref[0]
    lin = agg_ref[1]
    deg = lin[:, 32:33]
    maxdeg = jnp.maximum(deg, 1.0)
    degf = deg + 1.0
    zl = (jnp.dot(lin[:, 0:16], w1a_ref[...],
                  preferred_element_type=jnp.float32)
          + jnp.dot(lin[:, 16:30], w1b_ref[...],
                    preferred_element_type=jnp.float32)
          + jnp.dot(lin[:, 30:32], w1c_ref[...],
                    preferred_element_type=jnp.float32))
    h1l = jnp.maximum(zl / maxdeg + b1_ref[...], 0.0)
    t = (s1 + h1l) / degf
    m_ref[...] = (jnp.dot(t, w2_ref[...], preferred_element_type=jnp.float32)
                  + b2_ref[...])
    invd_ref[...] = jnp.broadcast_to(1.0 / degf, invd_ref.shape)


def _run_k4(agg, w1a, w1b, w1c, b1r, w2, b2r):
    return pl.pallas_call(
        _k4_body,
        grid=(N // BN,),
        in_specs=[
            pl.BlockSpec((NC, BN, H), lambda i: (0, i, 0)),
            pl.BlockSpec((16, H), lambda i: (0, 0)),
            pl.BlockSpec((14, H), lambda i: (0, 0)),
            pl.BlockSpec((2, H), lambda i: (0, 0)),
            pl.BlockSpec((1, H), lambda i: (0, 0)),
            pl.BlockSpec((H, H), lambda i: (0, 0)),
            pl.BlockSpec((1, H), lambda i: (0, 0)),
        ],
        out_specs=(pl.BlockSpec((BN, H), lambda i: (i, 0)),
                   pl.BlockSpec((BN, 8), lambda i: (i, 0))),
        out_shape=(jax.ShapeDtypeStruct((N, H), jnp.float32),
                   jax.ShapeDtypeStruct((N, 8), jnp.float32)),
    )(agg, w1a, w1b, w1c, b1r, w2, b2r)


def _k5_body(v, src, dst, part_out, accum, si0, si1, di, b0, b1, zbuf,
             sem0, sem1):
    c = lax.axis_index("c")
    s = lax.axis_index("s")
    wid = s * NC + c
    nmine = W_BASE + jnp.where(wid < W_EXTRA, 1, 0)
    npairs = nmine // 2
    tail = nmine - npairs * 2

    _zero_rows(zbuf, ZR)
    row0 = s * RPS
    for k in range(RPS // ZR):
        pltpu.sync_copy(zbuf, accum.at[pl.ds(row0 + k * ZR, ZR)])
    plsc.subcore_barrier()

    def cbase(k):
        return (k * NW + wid) * CHUNK

    def startg(k, si, buf, sem):
        pltpu.sync_copy(src.at[pl.ds(cbase(k), CHUNK)], si)
        pltpu.async_copy(v.at[si], buf, sem)

    def waitg(buf, sem):
        pltpu.make_async_copy(v.at[si0], buf, sem).wait()

    startg(0, si0, b0, sem0)

    def pair(j, _):
        k0 = 2 * j
        startg(k0 + 1, si1, b1, sem1)
        waitg(b0, sem0)
        pltpu.sync_copy(dst.at[pl.ds(cbase(k0), CHUNK)], di)
        pltpu.sync_copy(b0, accum.at[di], add=True)

        @pl.when(k0 + 2 < nmine)
        def _():
            startg(k0 + 2, si0, b0, sem0)

        waitg(b1, sem1)
        pltpu.sync_copy(dst.at[pl.ds(cbase(k0 + 1), CHUNK)], di)
        pltpu.sync_copy(b1, accum.at[di], add=True)
        return 0
    lax.fori_loop(0, npairs, pair, 0)

    @pl.when(tail == 1)
    def _():
        k = npairs * 2
        waitg(b0, sem0)
        pltpu.sync_copy(dst.at[pl.ds(cbase(k), CHUNK)], di)
        pltpu.sync_copy(b0, accum.at[di], add=True)
    plsc.subcore_barrier()

    pltpu.sync_copy(accum.at[pl.ds(row0, RPS)],
                    part_out.at[c, pl.ds(row0, RPS)])


_k5 = functools.partial(
    pl.kernel, _k5_body, mesh=_mesh,
    out_type=jax.ShapeDtypeStruct((NC, NP, H), jnp.float32),
    scratch_types=[
        pltpu.VMEM_SHARED((NP, H), jnp.float32),
        pltpu.VMEM((CHUNK,), jnp.int32),
        pltpu.VMEM((CHUNK,), jnp.int32),
        pltpu.VMEM((CHUNK,), jnp.int32),
        pltpu.VMEM((CHUNK, H), jnp.float32),
        pltpu.VMEM((CHUNK, H), jnp.float32),
        pltpu.VMEM((ZR, H), jnp.float32),
        pltpu.SemaphoreType.DMA,
        pltpu.SemaphoreType.DMA,
    ],
)()


def _k6_body(p_ref, m_ref, v_ref, invd_ref, out_ref):
    out_ref[...] = m_ref[...] + (p_ref[0] + p_ref[1] + v_ref[...]) \
        * invd_ref[:, 0:1]


def _run_k6(part, m, v, invd):
    return pl.pallas_call(
        _k6_body,
        grid=(N // BN,),
        in_specs=[
            pl.BlockSpec((NC, BN, H), lambda i: (0, i, 0)),
            pl.BlockSpec((BN, H), lambda i: (i, 0)),
            pl.BlockSpec((BN, H), lambda i: (i, 0)),
            pl.BlockSpec((BN, 8), lambda i: (i, 0)),
        ],
        out_specs=pl.BlockSpec((BN, H), lambda i: (i, 0)),
        out_shape=jax.ShapeDtypeStruct((N, H), jnp.float32),
    )(part, m, v, invd)


def _k7_body(p_ref, m_ref, v_ref, invd_ref, batch_ref, wc1_ref, bc1_ref,
             wc2_ref, bc2_ref, out_ref, acc, gcnt):
    i = pl.program_id(0)

    @pl.when(i == 0)
    def _():
        acc[...] = jnp.zeros_like(acc)
        gcnt[...] = jnp.zeros_like(gcnt)

    x3 = m_ref[...] + (p_ref[0] + p_ref[1] + v_ref[...]) \
        * invd_ref[:, 0:1]
    gids = lax.broadcasted_iota(jnp.int32, (G, BN), 0)
    mask = (gids == batch_ref[0]).astype(jnp.float32)
    acc[...] += jnp.dot(mask, x3, preferred_element_type=jnp.float32)
    gcnt[...] += jnp.broadcast_to(
        jnp.sum(mask, axis=1, keepdims=True), gcnt.shape)

    @pl.when(i == pl.num_programs(0) - 1)
    def _():
        pooled = acc[...] / jnp.maximum(gcnt[...], 1.0)
        h = jnp.maximum(
            jnp.dot(pooled, wc1_ref[...], preferred_element_type=jnp.float32)
            + bc1_ref[...], 0.0)
        out_ref[...] = (jnp.dot(h, wc2_ref[...],
                                preferred_element_type=jnp.float32)
                        + bc2_ref[...])


def _run_k7(part, m, v, invd, batch3d, wc1, bc1r, wc2, bc2r):
    return pl.pallas_call(
        _k7_body,
        grid=(N // BN,),
        in_specs=[
            pl.BlockSpec((NC, BN, H), lambda i: (0, i, 0)),
            pl.BlockSpec((BN, H), lambda i: (i, 0)),
            pl.BlockSpec((BN, H), lambda i: (i, 0)),
            pl.BlockSpec((BN, 8), lambda i: (i, 0)),
            pl.BlockSpec((1, 1, BN), lambda i: (i, 0, 0)),
            pl.BlockSpec((H, H), lambda i: (0, 0)),
            pl.BlockSpec((1, H), lambda i: (0, 0)),
            pl.BlockSpec((H, 10), lambda i: (0, 0)),
            pl.BlockSpec((1, 10), lambda i: (0, 0)),
        ],
        out_specs=pl.BlockSpec((G, 10), lambda i: (0, 0)),
        out_shape=jax.ShapeDtypeStruct((G, 10), jnp.float32),
        scratch_shapes=[pltpu.VMEM((G, H), jnp.float32),
                        pltpu.VMEM((G, H), jnp.float32)],
    )(part, m, v, invd, batch3d, wc1, bc1r, wc2, bc2r)


def kernel(edge_attr, dst_ports, tcp_flags, edge_index, batch, y,
           port_emb, tcp_emb, W1, b1, W2, b2, Wc1, bc1, Wc2, bc2):
    port_pad = jnp.pad(port_emb, ((0, 0), (0, H - 14)))
    src = edge_index[0]
    dst = edge_index[1]
    w1a = W1[:16]
    w1b = W1[16:30]
    w1c = W1[30:32]
    b1r = b1.reshape(1, H)
    b2r = b2.reshape(1, H)
    bc1r = bc1.reshape(1, H)
    bc2r = bc2.reshape(1, 10)
    tcp3d = tcp_flags.astype(jnp.int32).reshape(E // BE, 1, BE)
    batch3d = batch.astype(jnp.int32).reshape(N // BN, 1, BN)

    pe128 = _k1(port_pad, dst_ports)
    payA, payB = _run_k2(edge_attr, pe128, tcp3d, w1a, w1b, w1c, b1r, tcp_emb)
    agg = _k3(payA, payB, dst)
    m, invd = _run_k4(agg, w1a, w1b, w1c, b1r, W2, b2r)
    p1 = _k5(m, src, dst)
    x2 = _run_k6(p1, m, m, invd)
    p2 = _k5(x2, src, dst)
    return _run_k7(p2, m, x2, invd, batch3d, Wc1, bc1r, Wc2, bc2r)


# speedup vs baseline: 9.3766x; 1.0222x over previous
import functools

import jax
import jax.numpy as jnp
from jax import lax
from jax.experimental import pallas as pl
from jax.experimental.pallas import tpu as pltpu
from jax.experimental.pallas import tpu_sc as plsc

E = 320000
N = 10000
G = 64
H = 128
CHUNK = 128
NCHUNKS = E // CHUNK
NC, NS = 2, 16
NW = NC * NS
W_BASE, W_EXTRA = divmod(NCHUNKS, NW)
S_BASE, S_EXTRA = divmod(NCHUNKS, NS)
NP = 10240
RPS = NP // NS
ZR = 64
BE = 4000
BN = 2000

_mesh = plsc.VectorSubcoreMesh(core_axis_name="c", subcore_axis_name="s")


def _zero_rows(buf, nrows):
    def row(i, _):
        for j in range(buf.shape[1] // 16):
            buf[i, pl.ds(j * 16, 16)] = jnp.zeros((16,), jnp.float32)
        return 0
    lax.fori_loop(0, nrows, row, 0)


def _k1_body(port_pad, dst_ports, out, pi0, pi1, b0, b1, sem0, sem1):
    wid = lax.axis_index("s") * NC + lax.axis_index("c")
    nmine = W_BASE + jnp.where(wid < W_EXTRA, 1, 0)
    npairs = nmine // 2
    tail = nmine - npairs * 2

    def cbase(k):
        return (k * NW + wid) * CHUNK

    def startg(k, pi, buf, sem):
        pltpu.sync_copy(dst_ports.at[pl.ds(cbase(k), CHUNK)], pi)
        pltpu.async_copy(port_pad.at[pi], buf, sem)

    def waitg(buf, sem):
        pltpu.make_async_copy(port_pad.at[pi0], buf, sem).wait()

    startg(0, pi0, b0, sem0)

    def pair(j, _):
        k0 = 2 * j
        startg(k0 + 1, pi1, b1, sem1)
        waitg(b0, sem0)
        pltpu.sync_copy(b0, out.at[pl.ds(cbase(k0), CHUNK)])

        @pl.when(k0 + 2 < nmine)
        def _():
            startg(k0 + 2, pi0, b0, sem0)

        waitg(b1, sem1)
        pltpu.sync_copy(b1, out.at[pl.ds(cbase(k0 + 1), CHUNK)])
        return 0
    lax.fori_loop(0, npairs, pair, 0)

    @pl.when(tail == 1)
    def _():
        k = npairs * 2
        waitg(b0, sem0)
        pltpu.sync_copy(b0, out.at[pl.ds(cbase(k), CHUNK)])


_k1 = functools.partial(
    pl.kernel, _k1_body, mesh=_mesh,
    out_type=jax.ShapeDtypeStruct((E, H), jnp.float32),
    scratch_types=[
        pltpu.VMEM((CHUNK,), jnp.int32),
        pltpu.VMEM((CHUNK,), jnp.int32),
        pltpu.VMEM((CHUNK, H), jnp.float32),
        pltpu.VMEM((CHUNK, H), jnp.float32),
        pltpu.SemaphoreType.DMA,
        pltpu.SemaphoreType.DMA,
    ],
)()


def _k2_body(attr_ref, pe_ref, tcp_ref, w1a_ref, w1b_ref, w1c_ref, b1_ref,
             temb_ref, payA_ref, payB_ref):
    attr = attr_ref[...]
    pe = pe_ref[:, :14]
    t = tcp_ref[0]
    ohT = (lax.broadcasted_iota(jnp.int32, (256, BE), 0) == t
           ).astype(jnp.float32)
    te = lax.dot_general(ohT, temb_ref[...], (((0,), (0,)), ((), ())),
                         precision=lax.Precision.HIGHEST,
                         preferred_element_type=jnp.float32)
    z = (jnp.dot(attr, w1a_ref[...], preferred_element_type=jnp.float32)
         + jnp.dot(pe, w1b_ref[...], preferred_element_type=jnp.float32)
         + jnp.dot(te, w1c_ref[...], preferred_element_type=jnp.float32)
         + b1_ref[...])
    payA_ref[...] = jnp.maximum(z, 0.0)
    payB_ref[...] = jnp.concatenate(
        [attr, pe, te, jnp.ones((BE, 1), jnp.float32),
         jnp.zeros((BE, 95), jnp.float32)], axis=1)


def _run_k2(edge_attr, pe128, tcp3d, w1a, w1b, w1c, b1r, tcp_emb):
    return pl.pallas_call(
        _k2_body,
        grid=(E // BE,),
        in_specs=[
            pl.BlockSpec((BE, 16), lambda i: (i, 0)),
            pl.BlockSpec((BE, H), lambda i: (i, 0)),
            pl.BlockSpec((1, 1, BE), lambda i: (i, 0, 0)),
            pl.BlockSpec((16, H), lambda i: (0, 0)),
            pl.BlockSpec((14, H), lambda i: (0, 0)),
            pl.BlockSpec((2, H), lambda i: (0, 0)),
            pl.BlockSpec((1, H), lambda i: (0, 0)),
            pl.BlockSpec((256, 2), lambda i: (0, 0)),
        ],
        out_specs=(pl.BlockSpec((BE, H), lambda i: (i, 0)),
                   pl.BlockSpec((BE, H), lambda i: (i, 0))),
        out_shape=(jax.ShapeDtypeStruct((E, H), jnp.float32),
                   jax.ShapeDtypeStruct((E, H), jnp.float32)),
    )(edge_attr, pe128, tcp3d, w1a, w1b, w1c, b1r, tcp_emb)


def _k3_body(payA, payB, dst, out, accum, i0, i1, b0, b1, zbuf, sem0, sem1):
    c = lax.axis_index("c")
    s = lax.axis_index("s")
    nmine = S_BASE + jnp.where(s < S_EXTRA, 1, 0)
    npairs = nmine // 2
    tail = nmine - npairs * 2

    _zero_rows(zbuf, ZR)
    row0 = s * RPS
    for k in range(RPS // ZR):
        pltpu.sync_copy(zbuf, accum.at[pl.ds(row0 + k * ZR, ZR)])
    plsc.subcore_barrier()

    def cbase(k):
        return (k * NS + s) * CHUNK

    def start(k, buf, sem):
        @pl.when(c == 0)
        def _():
            pltpu.async_copy(payA.at[pl.ds(cbase(k), CHUNK)], buf, sem)

        @pl.when(c == 1)
        def _():
            pltpu.async_copy(payB.at[pl.ds(cbase(k), CHUNK)], buf, sem)

    def wait(buf, sem):
        pltpu.make_async_copy(payA.at[pl.ds(0, CHUNK)], buf, sem).wait()

    start(0, b0, sem0)

    def pair(j, _):
        k0 = 2 * j
        start(k0 + 1, b1, sem1)
        wait(b0, sem0)
        pltpu.sync_copy(dst.at[pl.ds(cbase(k0), CHUNK)], i0)
        pltpu.sync_copy(b0, accum.at[i0], add=True)

        @pl.when(k0 + 2 < nmine)
        def _():
            start(k0 + 2, b0, sem0)

        wait(b1, sem1)
        pltpu.sync_copy(dst.at[pl.ds(cbase(k0 + 1), CHUNK)], i1)
        pltpu.sync_copy(b1, accum.at[i1], add=True)
        return 0
    lax.fori_loop(0, npairs, pair, 0)

    @pl.when(tail == 1)
    def _():
        k = npairs * 2
        wait(b0, sem0)
        pltpu.sync_copy(dst.at[pl.ds(cbase(k), CHUNK)], i0)
        pltpu.sync_copy(b0, accum.at[i0], add=True)
    plsc.subcore_barrier()

    pltpu.sync_copy(accum.at[pl.ds(row0, RPS)],
                    out.at[c, pl.ds(row0, RPS)])


_k3 = functools.partial(
    pl.kernel, _k3_body, mesh=_mesh,
    out_type=jax.ShapeDtypeStruct((NC, NP, H), jnp.float32),
    scratch_types=[
        pltpu.VMEM_SHARED((NP, H), jnp.float32),
        pltpu.VMEM((CHUNK,), jnp.int32),
        pltpu.VMEM((CHUNK,), jnp.int32),
        pltpu.VMEM((CHUNK, H), jnp.float32),
        pltpu.VMEM((CHUNK, H), jnp.float32),
        pltpu.VMEM((ZR, H), jnp.float32),
        pltpu.SemaphoreType.DMA,
        pltpu.SemaphoreType.DMA,
    ],
)()


def _k4_body(agg_ref, w1a_ref, w1b_ref, w1c_ref, b1_ref, w2_ref,
             b2_ref, m_ref, invd_ref):
    s1 = agg_ref[0]
    lin = agg_ref[1]
    deg = lin[:, 32:33]
    maxdeg = jnp.maximum(deg, 1.0)
    degf = deg + 1.0
    zl = (jnp.dot(lin[:, 0:16], w1a_ref[...],
                  preferred_element_type=jnp.float32)
          + jnp.dot(lin[:, 16:30], w1b_ref[...],
                    preferred_element_type=jnp.float32)
          + jnp.dot(lin[:, 30:32], w1c_ref[...],
                    preferred_element_type=jnp.float32))
    h1l = jnp.maximum(zl / maxdeg + b1_ref[...], 0.0)
    t = (s1 + h1l) / degf
    m_ref[...] = (jnp.dot(t, w2_ref[...], preferred_element_type=jnp.float32)
                  + b2_ref[...])
    invd_ref[...] = jnp.broadcast_to(1.0 / degf, invd_ref.shape)


def _run_k4(agg, w1a, w1b, w1c, b1r, w2, b2r):
    return pl.pallas_call(
        _k4_body,
        grid=(N // BN,),
        in_specs=[
            pl.BlockSpec((NC, BN, H), lambda i: (0, i, 0)),
            pl.BlockSpec((16, H), lambda i: (0, 0)),
            pl.BlockSpec((14, H), lambda i: (0, 0)),
            pl.BlockSpec((2, H), lambda i: (0, 0)),
            pl.BlockSpec((1, H), lambda i: (0, 0)),
            pl.BlockSpec((H, H), lambda i: (0, 0)),
            pl.BlockSpec((1, H), lambda i: (0, 0)),
        ],
        out_specs=(pl.BlockSpec((BN, H), lambda i: (i, 0)),
                   pl.BlockSpec((BN, 8), lambda i: (i, 0))),
        out_shape=(jax.ShapeDtypeStruct((N, H), jnp.float32),
                   jax.ShapeDtypeStruct((N, 8), jnp.float32)),
    )(agg, w1a, w1b, w1c, b1r, w2, b2r)


def _k5_body(v, src, dst, part_out, accum, si0, si1, di, b0, b1, zbuf,
             sem0, sem1):
    c = lax.axis_index("c")
    s = lax.axis_index("s")
    wid = s * NC + c
    nmine = W_BASE + jnp.where(wid < W_EXTRA, 1, 0)
    npairs = nmine // 2
    tail = nmine - npairs * 2

    _zero_rows(zbuf, ZR)
    row0 = s * RPS
    for k in range(RPS // ZR):
        pltpu.sync_copy(zbuf, accum.at[pl.ds(row0 + k * ZR, ZR)])
    plsc.subcore_barrier()

    def cbase(k):
        return (k * NW + wid) * CHUNK

    def startg(k, si, buf, sem):
        pltpu.sync_copy(src.at[pl.ds(cbase(k), CHUNK)], si)
        pltpu.async_copy(v.at[si], buf, sem)

    def waitg(buf, sem):
        pltpu.make_async_copy(v.at[si0], buf, sem).wait()

    startg(0, si0, b0, sem0)

    def pair(j, _):
        k0 = 2 * j
        startg(k0 + 1, si1, b1, sem1)
        waitg(b0, sem0)
        pltpu.sync_copy(dst.at[pl.ds(cbase(k0), CHUNK)], di)
        pltpu.sync_copy(b0, accum.at[di], add=True)

        @pl.when(k0 + 2 < nmine)
        def _():
            startg(k0 + 2, si0, b0, sem0)

        waitg(b1, sem1)
        pltpu.sync_copy(dst.at[pl.ds(cbase(k0 + 1), CHUNK)], di)
        pltpu.sync_copy(b1, accum.at[di], add=True)
        return 0
    lax.fori_loop(0, npairs, pair, 0)

    @pl.when(tail == 1)
    def _():
        k = npairs * 2
        waitg(b0, sem0)
        pltpu.sync_copy(dst.at[pl.ds(cbase(k), CHUNK)], di)
        pltpu.sync_copy(b0, accum.at[di], add=True)
    plsc.subcore_barrier()

    pltpu.sync_copy(accum.at[pl.ds(row0, RPS)],
                    part_out.at[c, pl.ds(row0, RPS)])


_k5 = functools.partial(
    pl.kernel, _k5_body, mesh=_mesh,
    out_type=jax.ShapeDtypeStruct((NC, NP, H), jnp.float32),
    scratch_types=[
        pltpu.VMEM_SHARED((NP, H), jnp.float32),
        pltpu.VMEM((CHUNK,), jnp.int32),
        pltpu.VMEM((CHUNK,), jnp.int32),
        pltpu.VMEM((CHUNK,), jnp.int32),
        pltpu.VMEM((CHUNK, H), jnp.float32),
        pltpu.VMEM((CHUNK, H), jnp.float32),
        pltpu.VMEM((ZR, H), jnp.float32),
        pltpu.SemaphoreType.DMA,
        pltpu.SemaphoreType.DMA,
    ],
)()


def _k6_body(p_ref, m_ref, v_ref, invd_ref, out_ref):
    out_ref[...] = m_ref[...] + (p_ref[0] + p_ref[1] + v_ref[...]) \
        * invd_ref[:, 0:1]


def _run_k6(part, m, v, invd):
    return pl.pallas_call(
        _k6_body,
        grid=(N // BN,),
        in_specs=[
            pl.BlockSpec((NC, BN, H), lambda i: (0, i, 0)),
            pl.BlockSpec((BN, H), lambda i: (i, 0)),
            pl.BlockSpec((BN, H), lambda i: (i, 0)),
            pl.BlockSpec((BN, 8), lambda i: (i, 0)),
        ],
        out_specs=pl.BlockSpec((BN, H), lambda i: (i, 0)),
        out_shape=jax.ShapeDtypeStruct((N, H), jnp.float32),
    )(part, m, v, invd)


def _k7_body(p_ref, m_ref, v_ref, invd_ref, batch_ref, wc1_ref, bc1_ref,
             wc2_ref, bc2_ref, out_ref, acc, gcnt):
    i = pl.program_id(0)

    @pl.when(i == 0)
    def _():
        acc[...] = jnp.zeros_like(acc)
        gcnt[...] = jnp.zeros_like(gcnt)

    x3 = m_ref[...] + (p_ref[0] + p_ref[1] + v_ref[...]) \
        * invd_ref[:, 0:1]
    gids = lax.broadcasted_iota(jnp.int32, (G, BN), 0)
    mask = (gids == batch_ref[0]).astype(jnp.float32)
    acc[...] += jnp.dot(mask, x3, preferred_element_type=jnp.float32)
    gcnt[...] += jnp.broadcast_to(
        jnp.sum(mask, axis=1, keepdims=True), gcnt.shape)

    @pl.when(i == pl.num_programs(0) - 1)
    def _():
        pooled = acc[...] / jnp.maximum(gcnt[...], 1.0)
        h = jnp.maximum(
            jnp.dot(pooled, wc1_ref[...], preferred_element_type=jnp.float32)
            + bc1_ref[...], 0.0)
        out_ref[...] = (jnp.dot(h, wc2_ref[...],
                                preferred_element_type=jnp.float32)
                        + bc2_ref[...])


def _run_k7(part, m, v, invd, batch3d, wc1, bc1r, wc2, bc2r):
    return pl.pallas_call(
        _k7_body,
        grid=(N // BN,),
        in_specs=[
            pl.BlockSpec((NC, BN, H), lambda i: (0, i, 0)),
            pl.BlockSpec((BN, H), lambda i: (i, 0)),
            pl.BlockSpec((BN, H), lambda i: (i, 0)),
            pl.BlockSpec((BN, 8), lambda i: (i, 0)),
            pl.BlockSpec((1, 1, BN), lambda i: (i, 0, 0)),
            pl.BlockSpec((H, H), lambda i: (0, 0)),
            pl.BlockSpec((1, H), lambda i: (0, 0)),
            pl.BlockSpec((H, 10), lambda i: (0, 0)),
            pl.BlockSpec((1, 10), lambda i: (0, 0)),
        ],
        out_specs=pl.BlockSpec((G, 10), lambda i: (0, 0)),
        out_shape=jax.ShapeDtypeStruct((G, 10), jnp.float32),
        scratch_shapes=[pltpu.VMEM((G, H), jnp.float32),
                        pltpu.VMEM((G, H), jnp.float32)],
    )(part, m, v, invd, batch3d, wc1, bc1r, wc2, bc2r)


def kernel(edge_attr, dst_ports, tcp_flags, edge_index, batch, y,
           port_emb, tcp_emb, W1, b1, W2, b2, Wc1, bc1, Wc2, bc2):
    port_pad = jnp.pad(port_emb, ((0, 0), (0, H - 14)))
    src = edge_index[0]
    dst = edge_index[1]
    w1a = W1[:16]
    w1b = W1[16:30]
    w1c = W1[30:32]
    b1r = b1.reshape(1, H)
    b2r = b2.reshape(1, H)
    bc1r = bc1.reshape(1, H)
    bc2r = bc2.reshape(1, 10)
    tcp3d = tcp_flags.astype(jnp.int32).reshape(E // BE, 1, BE)
    batch3d = batch.astype(jnp.int32).reshape(N // BN, 1, BN)

    pe128 = _k1(port_pad, dst_ports)
    payA, payB = _run_k2(edge_attr, pe128, tcp3d, w1a, w1b, w1c, b1r, tcp_emb)
    agg = _k3(payA, payB, dst)
    m, invd = _run_k4(agg, w1a, w1b, w1c, b1r, W2, b2r)
    p1 = _k5(m, src, dst)
    x2 = _run_k6(p1, m, m, invd)
    p2 = _k5(x2, src, dst)
    return _run_k7(p2, m, x2, invd, batch3d, Wc1, bc1r, Wc2, bc2r)


# speedup vs baseline: 9.4723x; 1.0102x over previous
import functools

import jax
import jax.numpy as jnp
from jax import lax
from jax.experimental import pallas as pl
from jax.experimental.pallas import tpu as pltpu
from jax.experimental.pallas import tpu_sc as plsc

E = 320000
N = 10000
G = 64
H = 128
CHUNK = 128
NCHUNKS = E // CHUNK
NC, NS = 2, 16
NW = NC * NS
W_BASE, W_EXTRA = divmod(NCHUNKS, NW)
S_BASE, S_EXTRA = divmod(NCHUNKS, NS)
NP = 10240
RPS = NP // NS
ZR = 64
BE = 8000
BN = 2000

_mesh = plsc.VectorSubcoreMesh(core_axis_name="c", subcore_axis_name="s")


def _zero_rows(buf, nrows):
    def row(i, _):
        for j in range(buf.shape[1] // 16):
            buf[i, pl.ds(j * 16, 16)] = jnp.zeros((16,), jnp.float32)
        return 0
    lax.fori_loop(0, nrows, row, 0)


def _k1_body(port_pad, dst_ports, out, pi0, pi1, b0, b1, sem0, sem1):
    wid = lax.axis_index("s") * NC + lax.axis_index("c")
    nmine = W_BASE + jnp.where(wid < W_EXTRA, 1, 0)
    npairs = nmine // 2
    tail = nmine - npairs * 2

    def cbase(k):
        return (k * NW + wid) * CHUNK

    def startg(k, pi, buf, sem):
        pltpu.sync_copy(dst_ports.at[pl.ds(cbase(k), CHUNK)], pi)
        pltpu.async_copy(port_pad.at[pi], buf, sem)

    def waitg(buf, sem):
        pltpu.make_async_copy(port_pad.at[pi0], buf, sem).wait()

    startg(0, pi0, b0, sem0)

    def pair(j, _):
        k0 = 2 * j
        startg(k0 + 1, pi1, b1, sem1)
        waitg(b0, sem0)
        pltpu.sync_copy(b0, out.at[pl.ds(cbase(k0), CHUNK)])

        @pl.when(k0 + 2 < nmine)
        def _():
            startg(k0 + 2, pi0, b0, sem0)

        waitg(b1, sem1)
        pltpu.sync_copy(b1, out.at[pl.ds(cbase(k0 + 1), CHUNK)])
        return 0
    lax.fori_loop(0, npairs, pair, 0)

    @pl.when(tail == 1)
    def _():
        k = npairs * 2
        waitg(b0, sem0)
        pltpu.sync_copy(b0, out.at[pl.ds(cbase(k), CHUNK)])


_k1 = functools.partial(
    pl.kernel, _k1_body, mesh=_mesh,
    out_type=jax.ShapeDtypeStruct((E, H), jnp.float32),
    scratch_types=[
        pltpu.VMEM((CHUNK,), jnp.int32),
        pltpu.VMEM((CHUNK,), jnp.int32),
        pltpu.VMEM((CHUNK, H), jnp.float32),
        pltpu.VMEM((CHUNK, H), jnp.float32),
        pltpu.SemaphoreType.DMA,
        pltpu.SemaphoreType.DMA,
    ],
)()


def _k2_body(attr_ref, pe_ref, tcp_ref, w1a_ref, w1b_ref, w1c_ref, b1_ref,
             temb_ref, payA_ref, payB_ref):
    attr = attr_ref[...]
    pe = pe_ref[:, :14]
    t = tcp_ref[0]
    ohT = (lax.broadcasted_iota(jnp.int32, (256, BE), 0) == t
           ).astype(jnp.float32)
    te = lax.dot_general(ohT, temb_ref[...], (((0,), (0,)), ((), ())),
                         precision=lax.Precision.HIGHEST,
                         preferred_element_type=jnp.float32)
    z = (jnp.dot(attr, w1a_ref[...], preferred_element_type=jnp.float32)
         + jnp.dot(pe, w1b_ref[...], preferred_element_type=jnp.float32)
         + jnp.dot(te, w1c_ref[...], preferred_element_type=jnp.float32)
         + b1_ref[...])
    payA_ref[...] = jnp.maximum(z, 0.0)
    payB_ref[...] = jnp.concatenate(
        [attr, pe, te, jnp.ones((BE, 1), jnp.float32),
         jnp.zeros((BE, 95), jnp.float32)], axis=1)


def _run_k2(edge_attr, pe128, tcp3d, w1a, w1b, w1c, b1r, tcp_emb):
    return pl.pallas_call(
        _k2_body,
        grid=(E // BE,),
        in_specs=[
            pl.BlockSpec((BE, 16), lambda i: (i, 0)),
            pl.BlockSpec((BE, H), lambda i: (i, 0)),
            pl.BlockSpec((1, 1, BE), lambda i: (i, 0, 0)),
            pl.BlockSpec((16, H), lambda i: (0, 0)),
            pl.BlockSpec((14, H), lambda i: (0, 0)),
            pl.BlockSpec((2, H), lambda i: (0, 0)),
            pl.BlockSpec((1, H), lambda i: (0, 0)),
            pl.BlockSpec((256, 2), lambda i: (0, 0)),
        ],
        out_specs=(pl.BlockSpec((BE, H), lambda i: (i, 0)),
                   pl.BlockSpec((BE, H), lambda i: (i, 0))),
        out_shape=(jax.ShapeDtypeStruct((E, H), jnp.float32),
                   jax.ShapeDtypeStruct((E, H), jnp.float32)),
    )(edge_attr, pe128, tcp3d, w1a, w1b, w1c, b1r, tcp_emb)


def _k3_body(payA, payB, dst, out, accum, i0, i1, b0, b1, zbuf, sem0, sem1):
    c = lax.axis_index("c")
    s = lax.axis_index("s")
    nmine = S_BASE + jnp.where(s < S_EXTRA, 1, 0)
    npairs = nmine // 2
    tail = nmine - npairs * 2

    _zero_rows(zbuf, ZR)
    row0 = s * RPS
    for k in range(RPS // ZR):
        pltpu.sync_copy(zbuf, accum.at[pl.ds(row0 + k * ZR, ZR)])
    plsc.subcore_barrier()

    def cbase(k):
        return (k * NS + s) * CHUNK

    def start(k, buf, sem):
        @pl.when(c == 0)
        def _():
            pltpu.async_copy(payA.at[pl.ds(cbase(k), CHUNK)], buf, sem)

        @pl.when(c == 1)
        def _():
            pltpu.async_copy(payB.at[pl.ds(cbase(k), CHUNK)], buf, sem)

    def wait(buf, sem):
        pltpu.make_async_copy(payA.at[pl.ds(0, CHUNK)], buf, sem).wait()

    start(0, b0, sem0)

    def pair(j, _):
        k0 = 2 * j
        start(k0 + 1, b1, sem1)
        wait(b0, sem0)
        pltpu.sync_copy(dst.at[pl.ds(cbase(k0), CHUNK)], i0)
        pltpu.sync_copy(b0, accum.at[i0], add=True)

        @pl.when(k0 + 2 < nmine)
        def _():
            start(k0 + 2, b0, sem0)

        wait(b1, sem1)
        pltpu.sync_copy(dst.at[pl.ds(cbase(k0 + 1), CHUNK)], i1)
        pltpu.sync_copy(b1, accum.at[i1], add=True)
        return 0
    lax.fori_loop(0, npairs, pair, 0)

    @pl.when(tail == 1)
    def _():
        k = npairs * 2
        wait(b0, sem0)
        pltpu.sync_copy(dst.at[pl.ds(cbase(k), CHUNK)], i0)
        pltpu.sync_copy(b0, accum.at[i0], add=True)
    plsc.subcore_barrier()

    pltpu.sync_copy(accum.at[pl.ds(row0, RPS)],
                    out.at[c, pl.ds(row0, RPS)])


_k3 = functools.partial(
    pl.kernel, _k3_body, mesh=_mesh,
    out_type=jax.ShapeDtypeStruct((NC, NP, H), jnp.float32),
    scratch_types=[
        pltpu.VMEM_SHARED((NP, H), jnp.float32),
        pltpu.VMEM((CHUNK,), jnp.int32),
        pltpu.VMEM((CHUNK,), jnp.int32),
        pltpu.VMEM((CHUNK, H), jnp.float32),
        pltpu.VMEM((CHUNK, H), jnp.float32),
        pltpu.VMEM((ZR, H), jnp.float32),
        pltpu.SemaphoreType.DMA,
        pltpu.SemaphoreType.DMA,
    ],
)()


def _k4_body(agg_ref, w1a_ref, w1b_ref, w1c_ref, b1_ref, w2_ref,
             b2_ref, m_ref, invd_ref):
    s1 = agg_ref[0]
    lin = agg_ref[1]
    deg = lin[:, 32:33]
    maxdeg = jnp.maximum(deg, 1.0)
    degf = deg + 1.0
    zl = (jnp.dot(lin[:, 0:16], w1a_ref[...],
                  preferred_element_type=jnp.float32)
          + jnp.dot(lin[:, 16:30], w1b_ref[...],
                    preferred_element_type=jnp.float32)
          + jnp.dot(lin[:, 30:32], w1c_ref[...],
                    preferred_element_type=jnp.float32))
    h1l = jnp.maximum(zl / maxdeg + b1_ref[...], 0.0)
    t = (s1 + h1l) / degf
    m_ref[...] = (jnp.dot(t, w2_ref[...], preferred_element_type=jnp.float32)
                  + b2_ref[...])
    invd_ref[...] = jnp.broadcast_to(1.0 / degf, invd_ref.shape)


def _run_k4(agg, w1a, w1b, w1c, b1r, w2, b2r):
    return pl.pallas_call(
        _k4_body,
        grid=(N // BN,),
        in_specs=[
            pl.BlockSpec((NC, BN, H), lambda i: (0, i, 0)),
            pl.BlockSpec((16, H), lambda i: (0, 0)),
            pl.BlockSpec((14, H), lambda i: (0, 0)),
            pl.BlockSpec((2, H), lambda i: (0, 0)),
            pl.BlockSpec((1, H), lambda i: (0, 0)),
            pl.BlockSpec((H, H), lambda i: (0, 0)),
            pl.BlockSpec((1, H), lambda i: (0, 0)),
        ],
        out_specs=(pl.BlockSpec((BN, H), lambda i: (i, 0)),
                   pl.BlockSpec((BN, 8), lambda i: (i, 0))),
        out_shape=(jax.ShapeDtypeStruct((N, H), jnp.float32),
                   jax.ShapeDtypeStruct((N, 8), jnp.float32)),
    )(agg, w1a, w1b, w1c, b1r, w2, b2r)


def _k5_body(v, src, dst, part_out, accum, si0, si1, di, b0, b1, zbuf,
             sem0, sem1):
    c = lax.axis_index("c")
    s = lax.axis_index("s")
    wid = s * NC + c
    nmine = W_BASE + jnp.where(wid < W_EXTRA, 1, 0)
    npairs = nmine // 2
    tail = nmine - npairs * 2

    _zero_rows(zbuf, ZR)
    row0 = s * RPS
    for k in range(RPS // ZR):
        pltpu.sync_copy(zbuf, accum.at[pl.ds(row0 + k * ZR, ZR)])
    plsc.subcore_barrier()

    def cbase(k):
        return (k * NW + wid) * CHUNK

    def startg(k, si, buf, sem):
        pltpu.sync_copy(src.at[pl.ds(cbase(k), CHUNK)], si)
        pltpu.async_copy(v.at[si], buf, sem)

    def waitg(buf, sem):
        pltpu.make_async_copy(v.at[si0], buf, sem).wait()

    startg(0, si0, b0, sem0)

    def pair(j, _):
        k0 = 2 * j
        startg(k0 + 1, si1, b1, sem1)
        waitg(b0, sem0)
        pltpu.sync_copy(dst.at[pl.ds(cbase(k0), CHUNK)], di)
        pltpu.sync_copy(b0, accum.at[di], add=True)

        @pl.when(k0 + 2 < nmine)
        def _():
            startg(k0 + 2, si0, b0, sem0)

        waitg(b1, sem1)
        pltpu.sync_copy(dst.at[pl.ds(cbase(k0 + 1), CHUNK)], di)
        pltpu.sync_copy(b1, accum.at[di], add=True)
        return 0
    lax.fori_loop(0, npairs, pair, 0)

    @pl.when(tail == 1)
    def _():
        k = npairs * 2
        waitg(b0, sem0)
        pltpu.sync_copy(dst.at[pl.ds(cbase(k), CHUNK)], di)
        pltpu.sync_copy(b0, accum.at[di], add=True)
    plsc.subcore_barrier()

    pltpu.sync_copy(accum.at[pl.ds(row0, RPS)],
                    part_out.at[c, pl.ds(row0, RPS)])


_k5 = functools.partial(
    pl.kernel, _k5_body, mesh=_mesh,
    out_type=jax.ShapeDtypeStruct((NC, NP, H), jnp.float32),
    scratch_types=[
        pltpu.VMEM_SHARED((NP, H), jnp.float32),
        pltpu.VMEM((CHUNK,), jnp.int32),
        pltpu.VMEM((CHUNK,), jnp.int32),
        pltpu.VMEM((CHUNK,), jnp.int32),
        pltpu.VMEM((CHUNK, H), jnp.float32),
        pltpu.VMEM((CHUNK, H), jnp.float32),
        pltpu.VMEM((ZR, H), jnp.float32),
        pltpu.SemaphoreType.DMA,
        pltpu.SemaphoreType.DMA,
    ],
)()


def _k6_body(p_ref, m_ref, v_ref, invd_ref, out_ref):
    out_ref[...] = m_ref[...] + (p_ref[0] + p_ref[1] + v_ref[...]) \
        * invd_ref[:, 0:1]


def _run_k6(part, m, v, invd):
    return pl.pallas_call(
        _k6_body,
        grid=(N // BN,),
        in_specs=[
            pl.BlockSpec((NC, BN, H), lambda i: (0, i, 0)),
            pl.BlockSpec((BN, H), lambda i: (i, 0)),
            pl.BlockSpec((BN, H), lambda i: (i, 0)),
            pl.BlockSpec((BN, 8), lambda i: (i, 0)),
        ],
        out_specs=pl.BlockSpec((BN, H), lambda i: (i, 0)),
        out_shape=jax.ShapeDtypeStruct((N, H), jnp.float32),
    )(part, m, v, invd)


def _k7_body(p_ref, m_ref, v_ref, invd_ref, batch_ref, wc1_ref, bc1_ref,
             wc2_ref, bc2_ref, out_ref, acc, gcnt):
    i = pl.program_id(0)

    @pl.when(i == 0)
    def _():
        acc[...] = jnp.zeros_like(acc)
        gcnt[...] = jnp.zeros_like(gcnt)

    x3 = m_ref[...] + (p_ref[0] + p_ref[1] + v_ref[...]) \
        * invd_ref[:, 0:1]
    gids = lax.broadcasted_iota(jnp.int32, (G, BN), 0)
    mask = (gids == batch_ref[0]).astype(jnp.float32)
    acc[...] += jnp.dot(mask, x3, preferred_element_type=jnp.float32)
    gcnt[...] += jnp.broadcast_to(
        jnp.sum(mask, axis=1, keepdims=True), gcnt.shape)

    @pl.when(i == pl.num_programs(0) - 1)
    def _():
        pooled = acc[...] / jnp.maximum(gcnt[...], 1.0)
        h = jnp.maximum(
            jnp.dot(pooled, wc1_ref[...], preferred_element_type=jnp.float32)
            + bc1_ref[...], 0.0)
        out_ref[...] = (jnp.dot(h, wc2_ref[...],
                                preferred_element_type=jnp.float32)
                        + bc2_ref[...])


def _run_k7(part, m, v, invd, batch3d, wc1, bc1r, wc2, bc2r):
    return pl.pallas_call(
        _k7_body,
        grid=(N // BN,),
        in_specs=[
            pl.BlockSpec((NC, BN, H), lambda i: (0, i, 0)),
            pl.BlockSpec((BN, H), lambda i: (i, 0)),
            pl.BlockSpec((BN, H), lambda i: (i, 0)),
            pl.BlockSpec((BN, 8), lambda i: (i, 0)),
            pl.BlockSpec((1, 1, BN), lambda i: (i, 0, 0)),
            pl.BlockSpec((H, H), lambda i: (0, 0)),
            pl.BlockSpec((1, H), lambda i: (0, 0)),
            pl.BlockSpec((H, 10), lambda i: (0, 0)),
            pl.BlockSpec((1, 10), lambda i: (0, 0)),
        ],
        out_specs=pl.BlockSpec((G, 10), lambda i: (0, 0)),
        out_shape=jax.ShapeDtypeStruct((G, 10), jnp.float32),
        scratch_shapes=[pltpu.VMEM((G, H), jnp.float32),
                        pltpu.VMEM((G, H), jnp.float32)],
    )(part, m, v, invd, batch3d, wc1, bc1r, wc2, bc2r)


def kernel(edge_attr, dst_ports, tcp_flags, edge_index, batch, y,
           port_emb, tcp_emb, W1, b1, W2, b2, Wc1, bc1, Wc2, bc2):
    port_pad = jnp.pad(port_emb, ((0, 0), (0, H - 14)))
    src = edge_index[0]
    dst = edge_index[1]
    w1a = W1[:16]
    w1b = W1[16:30]
    w1c = W1[30:32]
    b1r = b1.reshape(1, H)
    b2r = b2.reshape(1, H)
    bc1r = bc1.reshape(1, H)
    bc2r = bc2.reshape(1, 10)
    tcp3d = tcp_flags.astype(jnp.int32).reshape(E // BE, 1, BE)
    batch3d = batch.astype(jnp.int32).reshape(N // BN, 1, BN)

    pe128 = _k1(port_pad, dst_ports)
    payA, payB = _run_k2(edge_attr, pe128, tcp3d, w1a, w1b, w1c, b1r, tcp_emb)
    agg = _k3(payA, payB, dst)
    m, invd = _run_k4(agg, w1a, w1b, w1c, b1r, W2, b2r)
    p1 = _k5(m, src, dst)
    x2 = _run_k6(p1, m, m, invd)
    p2 = _k5(x2, src, dst)
    return _run_k7(p2, m, x2, invd, batch3d, Wc1, bc1r, Wc2, bc2r)
